# baseline (speedup 1.0000x reference)
import functools

import jax
import jax.numpy as jnp
from jax.experimental import pallas as pl

_EPS = 1e-5


def _pick_block(total, target, mult=8):
    if total <= target:
        return total
    b = (target // mult) * mult
    while b >= mult:
        if total % b == 0:
            return b
        b -= mult
    return total


def _pack_rows(*rows):
    w = rows[0].shape[0]
    out = jnp.zeros((8, w), jnp.float32)
    for i, r in enumerate(rows):
        out = out.at[i].set(r)
    return out


def _ab(s1, s2, cnt, g, b):
    mu = s1 / cnt
    var = s2 / cnt - mu * mu
    inv = g / jnp.sqrt(var + _EPS)
    return inv, b - mu * inv


def _mm_bias_body(x_ref, w_ref, p_ref, o_ref):
    o_ref[...] = (
        jnp.dot(x_ref[...], w_ref[...], preferred_element_type=jnp.float32)
        + p_ref[0:1, :]
    )


def _mm_bias(x, W, b, bn=8192):
    N, K = x.shape
    H = W.shape[1]
    BN = _pick_block(N, bn)
    return pl.pallas_call(
        _mm_bias_body,
        grid=(N // BN,),
        in_specs=[
            pl.BlockSpec((BN, K), lambda i: (i, 0)),
            pl.BlockSpec((K, H), lambda i: (0, 0)),
            pl.BlockSpec((8, H), lambda i: (0, 0)),
        ],
        out_specs=pl.BlockSpec((BN, H), lambda i: (i, 0)),
        out_shape=jax.ShapeDtypeStruct((N, H), jnp.float32),
    )(x, W, _pack_rows(b))


def _edge_stats_body(g_ref, ea_ref, w1e_ref, o_ref):
    z1 = (
        g_ref[...]
        + ea_ref[:, 0:1] * w1e_ref[0:1, :]
        + ea_ref[:, 1:2] * w1e_ref[1:2, :]
    )
    s1 = jnp.sum(z1, axis=0, keepdims=True)
    s2 = jnp.sum(z1 * z1, axis=0, keepdims=True)
    contrib = jnp.concatenate(
        [s1, s2, jnp.zeros((6, s1.shape[1]), jnp.float32)], axis=0
    )

    @pl.when(pl.program_id(0) == 0)
    def _():
        o_ref[...] = jnp.zeros_like(o_ref)

    o_ref[...] += contrib


def _edge_stats(G, ea, W1e):
    E, H = G.shape
    BE = _pick_block(E, 8000)
    return pl.pallas_call(
        _edge_stats_body,
        grid=(E // BE,),
        in_specs=[
            pl.BlockSpec((BE, H), lambda i: (i, 0)),
            pl.BlockSpec((BE, 2), lambda i: (i, 0)),
            pl.BlockSpec((2, H), lambda i: (0, 0)),
        ],
        out_specs=pl.BlockSpec((8, H), lambda i: (0, 0)),
        out_shape=jax.ShapeDtypeStruct((8, H), jnp.float32),
    )(G, ea, W1e)


def _edge_pass2_body(g_ref, ea_ref, w1e_ref, p_ref, w2_ref, z2_ref, st_ref):
    z1 = (
        g_ref[...]
        + ea_ref[:, 0:1] * w1e_ref[0:1, :]
        + ea_ref[:, 1:2] * w1e_ref[1:2, :]
    )
    y = jax.nn.relu(z1 * p_ref[0:1, :] + p_ref[1:2, :])
    z2 = (
        jnp.dot(y, w2_ref[...], preferred_element_type=jnp.float32)
        + p_ref[2:3, :]
    )
    z2_ref[...] = z2
    s1 = jnp.sum(z2, axis=0, keepdims=True)
    s2 = jnp.sum(z2 * z2, axis=0, keepdims=True)
    contrib = jnp.concatenate(
        [s1, s2, jnp.zeros((6, s1.shape[1]), jnp.float32)], axis=0
    )

    @pl.when(pl.program_id(0) == 0)
    def _():
        st_ref[...] = jnp.zeros_like(st_ref)

    st_ref[...] += contrib


def _edge_pass2(G, ea, W1e, a1, b1n, W2, b2):
    E, H = G.shape
    BE = _pick_block(E, 8000)
    return pl.pallas_call(
        _edge_pass2_body,
        grid=(E // BE,),
        in_specs=[
            pl.BlockSpec((BE, H), lambda i: (i, 0)),
            pl.BlockSpec((BE, 2), lambda i: (i, 0)),
            pl.BlockSpec((2, H), lambda i: (0, 0)),
            pl.BlockSpec((8, H), lambda i: (0, 0)),
            pl.BlockSpec((H, H), lambda i: (0, 0)),
        ],
        out_specs=[
            pl.BlockSpec((BE, H), lambda i: (i, 0)),
            pl.BlockSpec((8, H), lambda i: (0, 0)),
        ],
        out_shape=[
            jax.ShapeDtypeStruct((E, H), jnp.float32),
            jax.ShapeDtypeStruct((8, H), jnp.float32),
        ],
    )(G, ea, W1e, _pack_rows(a1, b1n, b2), W2)


def _edge_m_body(z2_ref, p_ref, m_ref):
    m_ref[...] = jax.nn.relu(z2_ref[...] * p_ref[0:1, :] + p_ref[1:2, :])


def _edge_m(z2, a2, b2n):
    E, H = z2.shape
    BE = _pick_block(E, 8000)
    return pl.pallas_call(
        _edge_m_body,
        grid=(E // BE,),
        in_specs=[
            pl.BlockSpec((BE, H), lambda i: (i, 0)),
            pl.BlockSpec((8, H), lambda i: (0, 0)),
        ],
        out_specs=pl.BlockSpec((BE, H), lambda i: (i, 0)),
        out_shape=jax.ShapeDtypeStruct((E, H), jnp.float32),
    )(z2, _pack_rows(a2, b2n))


def _node1_body(x_ref, agg_ref, waa_ref, wab_ref, p_ref, z_ref, st_ref):
    z = (
        jnp.dot(x_ref[...], waa_ref[...], preferred_element_type=jnp.float32)
        + jnp.dot(agg_ref[...], wab_ref[...], preferred_element_type=jnp.float32)
        + p_ref[0:1, :]
    )
    z_ref[...] = z
    s1 = jnp.sum(z, axis=0, keepdims=True)
    s2 = jnp.sum(z * z, axis=0, keepdims=True)
    contrib = jnp.concatenate(
        [s1, s2, jnp.zeros((6, s1.shape[1]), jnp.float32)], axis=0
    )

    @pl.when(pl.program_id(0) == 0)
    def _():
        st_ref[...] = jnp.zeros_like(st_ref)

    st_ref[...] += contrib


def _node1(x, agg, Waa, Wab, ba):
    N, ND = x.shape
    H = agg.shape[1]
    BN = _pick_block(N, 8000)
    return pl.pallas_call(
        _node1_body,
        grid=(N // BN,),
        in_specs=[
            pl.BlockSpec((BN, ND), lambda i: (i, 0)),
            pl.BlockSpec((BN, H), lambda i: (i, 0)),
            pl.BlockSpec((ND, H), lambda i: (0, 0)),
            pl.BlockSpec((H, H), lambda i: (0, 0)),
            pl.BlockSpec((8, H), lambda i: (0, 0)),
        ],
        out_specs=[
            pl.BlockSpec((BN, H), lambda i: (i, 0)),
            pl.BlockSpec((8, H), lambda i: (0, 0)),
        ],
        out_shape=[
            jax.ShapeDtypeStruct((N, H), jnp.float32),
            jax.ShapeDtypeStruct((8, H), jnp.float32),
        ],
    )(x, agg, Waa, Wab, _pack_rows(ba))


def _node2_body(z_ref, x_ref, p_ref, wb_ref, o_ref):
    u = jax.nn.relu(z_ref[...] * p_ref[0:1, :] + p_ref[1:2, :])
    q = (
        jnp.dot(u, wb_ref[...], preferred_element_type=jnp.float32)
        + p_ref[2:3, 0:1]
    )
    comb = jax.nn.relu(q[:, 0:1])
    o_ref[...] = jnp.concatenate([x_ref[:, 0:10], comb], axis=1)


def _node2(z, x, aa, ban, Wb, bb):
    N, H = z.shape
    ND = x.shape[1]
    BN = _pick_block(N, 8000)
    bbrow = jnp.broadcast_to(bb.reshape(1, 1), (1, H)).reshape(H)
    return pl.pallas_call(
        _node2_body,
        grid=(N // BN,),
        in_specs=[
            pl.BlockSpec((BN, H), lambda i: (i, 0)),
            pl.BlockSpec((BN, ND), lambda i: (i, 0)),
            pl.BlockSpec((8, H), lambda i: (0, 0)),
            pl.BlockSpec((H, 1), lambda i: (0, 0)),
        ],
        out_specs=pl.BlockSpec((BN, ND), lambda i: (i, 0)),
        out_shape=jax.ShapeDtypeStruct((N, ND), jnp.float32),
    )(z, x, _pack_rows(aa, ban, bbrow), Wb)


def _p1_body(x_ref, w_ref, p_ref, z_ref, st_ref):
    z = (
        jnp.dot(x_ref[...], w_ref[...], preferred_element_type=jnp.float32)
        + p_ref[0:1, :]
    )
    z_ref[...] = z
    s1 = jnp.sum(z, axis=0, keepdims=True)
    s2 = jnp.sum(z * z, axis=0, keepdims=True)
    contrib = jnp.concatenate(
        [s1, s2, jnp.zeros((6, s1.shape[1]), jnp.float32)], axis=0
    )

    @pl.when(pl.program_id(0) == 0)
    def _():
        st_ref[...] = jnp.zeros_like(st_ref)

    st_ref[...] += contrib


def _p1(x, W, b):
    N, ND = x.shape
    H = W.shape[1]
    BN = _pick_block(N, 8000)
    return pl.pallas_call(
        _p1_body,
        grid=(N // BN,),
        in_specs=[
            pl.BlockSpec((BN, ND), lambda i: (i, 0)),
            pl.BlockSpec((ND, H), lambda i: (0, 0)),
            pl.BlockSpec((8, H), lambda i: (0, 0)),
        ],
        out_specs=[
            pl.BlockSpec((BN, H), lambda i: (i, 0)),
            pl.BlockSpec((8, H), lambda i: (0, 0)),
        ],
        out_shape=[
            jax.ShapeDtypeStruct((N, H), jnp.float32),
            jax.ShapeDtypeStruct((8, H), jnp.float32),
        ],
    )(x, W, _pack_rows(b))


def _p2_body(z_ref, p_ref, w_ref, q_ref, st_ref):
    pact = jax.nn.relu(z_ref[...] * p_ref[0:1, :] + p_ref[1:2, :])
    q = (
        jnp.dot(pact, w_ref[...], preferred_element_type=jnp.float32)
        + p_ref[2:3, 0:1]
    )
    q = q[:, 0:1]
    q_ref[...] = q
    s1 = jnp.sum(q)
    s2 = jnp.sum(q * q)
    H = p_ref.shape[1]
    contrib = jnp.concatenate(
        [
            jnp.full((1, H), s1, jnp.float32),
            jnp.full((1, H), s2, jnp.float32),
            jnp.zeros((6, H), jnp.float32),
        ],
        axis=0,
    )

    @pl.when(pl.program_id(0) == 0)
    def _():
        st_ref[...] = jnp.zeros_like(st_ref)

    st_ref[...] += contrib


def _p2(z, ap, bpn, W, b):
    N, H = z.shape
    BN = _pick_block(N, 8000)
    bbrow = jnp.broadcast_to(b.reshape(1, 1), (1, H)).reshape(H)
    return pl.pallas_call(
        _p2_body,
        grid=(N // BN,),
        in_specs=[
            pl.BlockSpec((BN, H), lambda i: (i, 0)),
            pl.BlockSpec((8, H), lambda i: (0, 0)),
            pl.BlockSpec((H, 1), lambda i: (0, 0)),
        ],
        out_specs=[
            pl.BlockSpec((BN, 1), lambda i: (i, 0)),
            pl.BlockSpec((8, H), lambda i: (0, 0)),
        ],
        out_shape=[
            jax.ShapeDtypeStruct((N, 1), jnp.float32),
            jax.ShapeDtypeStruct((8, H), jnp.float32),
        ],
    )(z, _pack_rows(ap, bpn, bbrow), W)


def _p3_body(q_ref, p_ref, o_ref):
    o_ref[...] = jax.nn.relu(q_ref[...] * p_ref[0:1, 0:1] + p_ref[1:2, 0:1])


def _p3(q, aq, bqn, H=32):
    N = q.shape[0]
    BN = _pick_block(N, 20000)
    arow = jnp.broadcast_to(aq.reshape(1, 1), (1, H)).reshape(H)
    brow = jnp.broadcast_to(bqn.reshape(1, 1), (1, H)).reshape(H)
    return pl.pallas_call(
        _p3_body,
        grid=(N // BN,),
        in_specs=[
            pl.BlockSpec((BN, 1), lambda i: (i, 0)),
            pl.BlockSpec((8, H), lambda i: (0, 0)),
        ],
        out_specs=pl.BlockSpec((BN, 1), lambda i: (i, 0)),
        out_shape=jax.ShapeDtypeStruct((N, 1), jnp.float32),
    )(q, _pack_rows(arow, brow))


def kernel(x, edge_index, edge_attr, W1, b1, g1, be1, W2, b2, g2, be2,
           Wa, ba, ga, bea, Wb, bb, Wp1, bp1, gp1, bep1, Wp2, bp2, gp2, bep2):
    src = edge_index[0]
    dst = edge_index[1]
    N, ND = x.shape
    E = src.shape[0]
    W1a, W1e = W1[:ND], W1[ND:]
    Waa, Wab = Wa[:ND], Wa[ND:]

    for _ in range(3):
        xw = _mm_bias(x, W1a, b1)
        G = jnp.take(xw, src, axis=0)
        st = _edge_stats(G, edge_attr, W1e)
        a1, b1n = _ab(st[0], st[1], E, g1, be1)
        z2, st2 = _edge_pass2(G, edge_attr, W1e, a1, b1n, W2, b2)
        a2, b2n = _ab(st2[0], st2[1], E, g2, be2)
        m = _edge_m(z2, a2, b2n)
        agg = jax.ops.segment_max(m, dst, num_segments=N)
        agg = jnp.where(jnp.isfinite(agg), agg, 0.0)
        z, st3 = _node1(x, agg, Waa, Wab, ba)
        aa, ban = _ab(st3[0], st3[1], N, ga, bea)
        x = _node2(z, x, aa, ban, Wb, bb)

    zp, st4 = _p1(x, Wp1, bp1)
    ap, bpn = _ab(st4[0], st4[1], N, gp1, bep1)
    q, st5 = _p2(zp, ap, bpn, Wp2, bp2)
    aq, bqn = _ab(st5[0, 0:1], st5[1, 0:1], N, gp2, bep2)
    out = _p3(q, aq, bqn)
    return out
